# Initial kernel scaffold; baseline (speedup 1.0000x reference)
import jax
import jax.numpy as jnp
from jax.experimental import pallas as pl


def kernel(bloom_levels, bloom_dim_logits):
    raise NotImplementedError("write your pallas kernel here")


# speedup vs baseline: 1.6252x; 1.6252x over previous
import jax
import jax.numpy as jnp
from jax import lax
from jax.experimental import pallas as pl
from jax.experimental.pallas import tpu as pltpu
from jax.experimental.pallas import tpu_sc as plsc

_B = 16384
_K = 6
_NW = 32
_QPW = _B // _NW
_FPW = _QPW * _K
_DIMS = (64.0, 128.0, 256.0, 384.0, 512.0, 768.0)


def _dims_vec(iota):
    d = jnp.zeros((16,), jnp.float32)
    for i, v in enumerate(_DIMS):
        d = jnp.where(iota == i, jnp.float32(v), d)
    return d


def _sc_body(lvl_hbm, tab_hbm, sel_hbm, sdim_hbm, lg_hbm, pr_hbm, io_hbm,
             cnt_hbm, idx_v, tab_v, ptab, stab, dtab, itab,
             sbuf, dbuf, lbuf, pbuf, iobuf, cnt_v):
    wid = lax.axis_index("s") * 2 + lax.axis_index("c")
    base = wid * _QPW

    pltpu.sync_copy(lvl_hbm.at[pl.ds(base, _QPW)], idx_v)
    pltpu.sync_copy(tab_hbm, tab_v.at[pl.ds(0, _K * _K)])

    iota = lax.iota(jnp.int32, 16)
    valid = iota < _K
    iota_c = jnp.where(valid, iota, _K - 1)
    dims = _dims_vec(iota)

    sdim_vec = jnp.zeros((16,), jnp.float32)
    itab_vec = jnp.zeros((16,), jnp.int32)
    for l in range(_K):
        lsplat = jnp.full((16,), l, jnp.int32)
        row = plsc.load_gather(tab_v, [lsplat * _K + iota_c])
        m = jnp.max(jnp.where(valid, row, jnp.float32(-3e38)))
        e = jnp.where(valid, jnp.exp(row - m), jnp.float32(0.0))
        p = e / jnp.sum(e)
        pm = jnp.max(jnp.where(valid, p, jnp.float32(-1.0)))
        first = plsc.all_reduce_ffs((p == pm) & valid)
        onehot = jnp.where(iota == first, jnp.float32(1.0), jnp.float32(0.0))
        sel = (onehot - p) + p
        sdim_l = jnp.sum(sel * dims)
        fidx = lsplat * _K + iota
        plsc.store_scatter(ptab, [fidx], p, mask=valid)
        plsc.store_scatter(stab, [fidx], sel, mask=valid)
        sdim_vec = jnp.where(iota == l, sdim_l, sdim_vec)
        itab_vec = jnp.where(iota == l, first, itab_vec)
    dtab[pl.ds(0, 16)] = sdim_vec
    itab[pl.ds(0, 16)] = itab_vec

    def q_body(c, cnt):
        lvl = idx_v[pl.ds(c * 16, 16)]
        dbuf[pl.ds(c * 16, 16)] = plsc.load_gather(dtab, [lvl])
        iobuf[pl.ds(c * 16, 16)] = plsc.load_gather(itab, [lvl])
        for l in range(_K):
            cl = plsc.all_reduce_population_count(lvl == l)
            cnt = jnp.where(iota == l, cnt + cl, cnt)
        return cnt

    cnt = lax.fori_loop(0, _QPW // 16, q_body, jnp.zeros((16,), jnp.int32))

    def f_body(c, _):
        fi = c * 16 + iota
        q = lax.shift_right_logical(fi * 43691, 18)
        k = fi - q * _K
        fidx = plsc.load_gather(idx_v, [q]) * _K + k
        lbuf[pl.ds(c * 16, 16)] = plsc.load_gather(tab_v, [fidx])
        pbuf[pl.ds(c * 16, 16)] = plsc.load_gather(ptab, [fidx])
        sbuf[pl.ds(c * 16, 16)] = plsc.load_gather(stab, [fidx])
        return 0

    lax.fori_loop(0, _FPW // 16, f_body, 0)

    cnt_v[...] = cnt
    fbase = wid * _FPW
    pltpu.sync_copy(sbuf, sel_hbm.at[pl.ds(fbase, _FPW)])
    pltpu.sync_copy(lbuf, lg_hbm.at[pl.ds(fbase, _FPW)])
    pltpu.sync_copy(pbuf, pr_hbm.at[pl.ds(fbase, _FPW)])
    pltpu.sync_copy(dbuf, sdim_hbm.at[pl.ds(base, _QPW)])
    pltpu.sync_copy(iobuf, io_hbm.at[pl.ds(base, _QPW)])
    pltpu.sync_copy(cnt_v, cnt_hbm.at[wid])


def _sc_call(bloom_levels, bloom_dim_logits):
    mesh = plsc.VectorSubcoreMesh(core_axis_name="c", subcore_axis_name="s")
    out_type = (
        jax.ShapeDtypeStruct((_B * _K,), jnp.float32),
        jax.ShapeDtypeStruct((_B,), jnp.float32),
        jax.ShapeDtypeStruct((_B * _K,), jnp.float32),
        jax.ShapeDtypeStruct((_B * _K,), jnp.float32),
        jax.ShapeDtypeStruct((_B,), jnp.int32),
        jax.ShapeDtypeStruct((_NW, 16), jnp.int32),
    )
    scratch = [
        pltpu.VMEM((_QPW,), jnp.int32),
        pltpu.VMEM((128,), jnp.float32),
        pltpu.VMEM((128,), jnp.float32),
        pltpu.VMEM((128,), jnp.float32),
        pltpu.VMEM((128,), jnp.float32),
        pltpu.VMEM((128,), jnp.int32),
        pltpu.VMEM((_FPW,), jnp.float32),
        pltpu.VMEM((_QPW,), jnp.float32),
        pltpu.VMEM((_FPW,), jnp.float32),
        pltpu.VMEM((_FPW,), jnp.float32),
        pltpu.VMEM((_QPW,), jnp.int32),
        pltpu.VMEM((16,), jnp.int32),
    ]
    return pl.kernel(
        _sc_body, out_type=out_type, mesh=mesh, scratch_types=scratch,
        compiler_params=pltpu.CompilerParams(needs_layout_passes=False),
    )(bloom_levels, bloom_dim_logits)


def _tc_body(tab_ref, cnt_ref, avg_ref, ent_ref, td_ref):
    lg = tab_ref[...]
    c6 = jnp.sum(cnt_ref[...], axis=0, keepdims=True)[:, :_K].astype(jnp.float32)
    m = jnp.max(lg, axis=1, keepdims=True)
    e = jnp.exp(lg - m)
    p = e / jnp.sum(e, axis=1, keepdims=True)
    iota2d = lax.broadcasted_iota(jnp.int32, (_K, _K), 1)
    dims2d = jnp.zeros((_K, _K), jnp.float32)
    for i, v in enumerate(_DIMS):
        dims2d = jnp.where(iota2d == i, jnp.float32(v), dims2d)
    td_ref[...] = jnp.sum(p * dims2d, axis=1, keepdims=True)
    ent_l = -jnp.sum(p * jnp.log(p + 1e-10), axis=1, keepdims=True)
    pm = jnp.max(p, axis=1, keepdims=True)
    first = jnp.min(jnp.where(p == pm, iota2d, _K), axis=1, keepdims=True)
    hard = (iota2d == first).astype(jnp.float32)
    sdim_l = jnp.sum(((hard - p) + p) * dims2d, axis=1, keepdims=True)
    inv_b = jnp.float32(1.0 / _B)
    avg_ref[...] = jnp.dot(c6, sdim_l, preferred_element_type=jnp.float32) * inv_b
    ent_ref[...] = jnp.dot(c6, ent_l, preferred_element_type=jnp.float32) * inv_b


def _tc_call(bloom_dim_logits, counts):
    return pl.pallas_call(
        _tc_body,
        out_shape=(
            jax.ShapeDtypeStruct((1, 1), jnp.float32),
            jax.ShapeDtypeStruct((1, 1), jnp.float32),
            jax.ShapeDtypeStruct((_K, 1), jnp.float32),
        ),
    )(bloom_dim_logits, counts)


def kernel(bloom_levels, bloom_dim_logits):
    lvl = bloom_levels.astype(jnp.int32)
    tab = bloom_dim_logits.astype(jnp.float32)
    sel_f, sdim, lg_f, pr_f, idx, counts = _sc_call(lvl, tab.reshape(_K * _K))
    avg11, ent11, td61 = _tc_call(tab, counts)
    selection = sel_f.reshape(_B, _K)
    logits = lg_f.reshape(_B, _K)
    probs = pr_f.reshape(_B, _K)
    avg_dim = avg11.reshape(())
    entropy = ent11.reshape(())
    table_dims = td61.reshape(_K)
    return (selection, sdim, avg_dim, entropy, table_dims, logits, probs, idx)


# speedup vs baseline: 1.7028x; 1.0478x over previous
import jax
import jax.numpy as jnp
from jax import lax
from jax.experimental import pallas as pl
from jax.experimental.pallas import tpu as pltpu
from jax.experimental.pallas import tpu_sc as plsc

_B = 16384
_K = 6
_NW = 32
_QPW = _B // _NW
_FPW = _QPW * _K
_DIMS = (64.0, 128.0, 256.0, 384.0, 512.0, 768.0)


def _dims_vec(iota):
    d = jnp.zeros((16,), jnp.float32)
    for i, v in enumerate(_DIMS):
        d = jnp.where(iota == i, jnp.float32(v), d)
    return d


def _div6(x):
    return lax.shift_right_logical(x * 43691, 18)


def _sc_body(lvl_hbm, tab_hbm, sel_hbm, sdim_hbm, lg_hbm, pr_hbm, io_hbm,
             idx_v, tab_v, ptab, stab, dtab, itab,
             sbuf, dbuf, lbuf, pbuf, iobuf, s0, s1, s2, s3, s4):
    wid = lax.axis_index("s") * 2 + lax.axis_index("c")
    base = wid * _QPW

    h_idx = pltpu.async_copy(lvl_hbm.at[pl.ds(base, _QPW)], idx_v, s0)
    h_tab = pltpu.async_copy(tab_hbm, tab_v.at[pl.ds(0, _K * _K)], s1)
    h_idx.wait()
    h_tab.wait()

    iota = lax.iota(jnp.int32, 16)
    valid = iota < _K
    iota_c = jnp.where(valid, iota, _K - 1)
    dims = _dims_vec(iota)

    sdim_vec = jnp.zeros((16,), jnp.float32)
    itab_vec = jnp.zeros((16,), jnp.int32)
    for l in range(_K):
        lsplat = jnp.full((16,), l, jnp.int32)
        row = plsc.load_gather(tab_v, [lsplat * _K + iota_c])
        m = jnp.max(jnp.where(valid, row, jnp.float32(-3e38)))
        e = jnp.where(valid, jnp.exp(row - m), jnp.float32(0.0))
        p = e / jnp.sum(e)
        pm = jnp.max(jnp.where(valid, p, jnp.float32(-1.0)))
        first = plsc.all_reduce_ffs((p == pm) & valid)
        onehot = jnp.where(iota == first, jnp.float32(1.0), jnp.float32(0.0))
        sel = (onehot - p) + p
        sdim_l = jnp.sum(sel * dims)
        fidx = lsplat * _K + iota
        plsc.store_scatter(ptab, [fidx], p, mask=valid)
        plsc.store_scatter(stab, [fidx], sel, mask=valid)
        sdim_vec = jnp.where(iota == l, sdim_l, sdim_vec)
        itab_vec = jnp.where(iota == l, first, itab_vec)
    dtab[pl.ds(0, 16)] = sdim_vec
    itab[pl.ds(0, 16)] = itab_vec

    def q_body(c, _):
        lvl = idx_v[pl.ds(c * 16, 16)]
        dbuf[pl.ds(c * 16, 16)] = plsc.load_gather(dtab, [lvl])
        iobuf[pl.ds(c * 16, 16)] = plsc.load_gather(itab, [lvl])
        return 0

    lax.fori_loop(0, _QPW // 16, q_body, 0)

    qoff = [_div6(u * 16 + iota) for u in range(3)]
    kk = [(u * 16 + iota) - qoff[u] * _K for u in range(3)]

    def f_body(c, _):
        fb = c * 48
        qb = c * 8
        for u in range(3):
            fidx = plsc.load_gather(idx_v, [qb + qoff[u]]) * _K + kk[u]
            lbuf[pl.ds(fb + u * 16, 16)] = plsc.load_gather(tab_v, [fidx])
            pbuf[pl.ds(fb + u * 16, 16)] = plsc.load_gather(ptab, [fidx])
            sbuf[pl.ds(fb + u * 16, 16)] = plsc.load_gather(stab, [fidx])
        return 0

    lax.fori_loop(0, _FPW // 48, f_body, 0)

    fbase = wid * _FPW
    h0 = pltpu.async_copy(sbuf, sel_hbm.at[pl.ds(fbase, _FPW)], s0)
    h1 = pltpu.async_copy(lbuf, lg_hbm.at[pl.ds(fbase, _FPW)], s1)
    h2 = pltpu.async_copy(pbuf, pr_hbm.at[pl.ds(fbase, _FPW)], s2)
    h3 = pltpu.async_copy(dbuf, sdim_hbm.at[pl.ds(base, _QPW)], s3)
    h4 = pltpu.async_copy(iobuf, io_hbm.at[pl.ds(base, _QPW)], s4)
    h0.wait()
    h1.wait()
    h2.wait()
    h3.wait()
    h4.wait()


def _sc_call(bloom_levels, tab_flat):
    mesh = plsc.VectorSubcoreMesh(core_axis_name="c", subcore_axis_name="s")
    out_type = (
        jax.ShapeDtypeStruct((_B * _K,), jnp.float32),
        jax.ShapeDtypeStruct((_B,), jnp.float32),
        jax.ShapeDtypeStruct((_B * _K,), jnp.float32),
        jax.ShapeDtypeStruct((_B * _K,), jnp.float32),
        jax.ShapeDtypeStruct((_B,), jnp.int32),
    )
    scratch = [
        pltpu.VMEM((_QPW,), jnp.int32),
        pltpu.VMEM((128,), jnp.float32),
        pltpu.VMEM((128,), jnp.float32),
        pltpu.VMEM((128,), jnp.float32),
        pltpu.VMEM((128,), jnp.float32),
        pltpu.VMEM((128,), jnp.int32),
        pltpu.VMEM((_FPW,), jnp.float32),
        pltpu.VMEM((_QPW,), jnp.float32),
        pltpu.VMEM((_FPW,), jnp.float32),
        pltpu.VMEM((_FPW,), jnp.float32),
        pltpu.VMEM((_QPW,), jnp.int32),
        pltpu.SemaphoreType.DMA,
        pltpu.SemaphoreType.DMA,
        pltpu.SemaphoreType.DMA,
        pltpu.SemaphoreType.DMA,
        pltpu.SemaphoreType.DMA,
    ]
    return pl.kernel(
        _sc_body, out_type=out_type, mesh=mesh, scratch_types=scratch,
        compiler_params=pltpu.CompilerParams(needs_layout_passes=False),
    )(bloom_levels, tab_flat)


def _tc_body(tab_ref, lvl_ref, avg_ref, ent_ref, td_ref):
    lg = tab_ref[...]
    lvl2d = lvl_ref[...]
    m = jnp.max(lg, axis=1, keepdims=True)
    e = jnp.exp(lg - m)
    p = e / jnp.sum(e, axis=1, keepdims=True)
    iota2d = lax.broadcasted_iota(jnp.int32, (_K, _K), 1)
    dims2d = jnp.zeros((_K, _K), jnp.float32)
    for i, v in enumerate(_DIMS):
        dims2d = jnp.where(iota2d == i, jnp.float32(v), dims2d)
    td_ref[...] = jnp.sum(p * dims2d, axis=1, keepdims=True)
    ent_l = -jnp.sum(p * jnp.log(p + 1e-10), axis=1, keepdims=True)
    pm = jnp.max(p, axis=1, keepdims=True)
    first = jnp.min(jnp.where(p == pm, iota2d, _K), axis=1, keepdims=True)
    hard = (iota2d == first).astype(jnp.float32)
    sdim_l = jnp.sum(((hard - p) + p) * dims2d, axis=1, keepdims=True)
    iota16 = lax.broadcasted_iota(jnp.int32, (1, _K), 1)
    c6 = jnp.zeros((1, _K), jnp.float32)
    for l in range(_K):
        cl = jnp.sum((lvl2d == l).astype(jnp.float32))
        c6 = jnp.where(iota16 == l, cl, c6)
    inv_b = jnp.float32(1.0 / _B)
    avg_ref[...] = jnp.dot(c6, sdim_l, preferred_element_type=jnp.float32) * inv_b
    ent_ref[...] = jnp.dot(c6, ent_l, preferred_element_type=jnp.float32) * inv_b


def _tc_call(bloom_dim_logits, lvl2d):
    return pl.pallas_call(
        _tc_body,
        out_shape=(
            jax.ShapeDtypeStruct((1, 1), jnp.float32),
            jax.ShapeDtypeStruct((1, 1), jnp.float32),
            jax.ShapeDtypeStruct((_K, 1), jnp.float32),
        ),
    )(bloom_dim_logits, lvl2d)


def kernel(bloom_levels, bloom_dim_logits):
    lvl = bloom_levels.astype(jnp.int32)
    tab = bloom_dim_logits.astype(jnp.float32)
    sel_f, sdim, lg_f, pr_f, idx = _sc_call(lvl, tab.reshape(_K * _K))
    avg11, ent11, td61 = _tc_call(tab, lvl.reshape(128, 128))
    selection = sel_f.reshape(_B, _K)
    logits = lg_f.reshape(_B, _K)
    probs = pr_f.reshape(_B, _K)
    avg_dim = avg11.reshape(())
    entropy = ent11.reshape(())
    table_dims = td61.reshape(_K)
    return (selection, sdim, avg_dim, entropy, table_dims, logits, probs, idx)


# speedup vs baseline: 2.2302x; 1.3097x over previous
import jax
import jax.numpy as jnp
from jax import lax
from jax.experimental import pallas as pl
from jax.experimental.pallas import tpu as pltpu
from jax.experimental.pallas import tpu_sc as plsc

_B = 16384
_K = 6
_NW = 32
_QPW = _B // _NW
_DIMS = (64.0, 128.0, 256.0, 384.0, 512.0, 768.0)
_GRID = 8
_QPG = _B // _GRID


def _dims_vec(iota):
    d = jnp.zeros((16,), jnp.float32)
    for i, v in enumerate(_DIMS):
        d = jnp.where(iota == i, jnp.float32(v), d)
    return d


def _sc_body(lvl_hbm, tab_hbm, sel_hbm, sdim_hbm, lg_hbm, pr_hbm, io_hbm,
             idx_v, tab_v, ptab, stab, dtab, itab,
             sbuf, dbuf, lbuf, pbuf, iobuf, s0, s1, s2, s3):
    wid = lax.axis_index("s") * 2 + lax.axis_index("c")
    base = wid * _QPW

    h_idx = pltpu.async_copy(lvl_hbm.at[pl.ds(base, _QPW)], idx_v, s0)
    h_tab = pltpu.async_copy(tab_hbm, tab_v.at[pl.ds(0, _K * _K)], s1)
    h_idx.wait()
    h_tab.wait()

    iota = lax.iota(jnp.int32, 16)
    valid = iota < _K
    iota_c = jnp.where(valid, iota, _K - 1)
    dims = _dims_vec(iota)

    sdim_vec = jnp.zeros((16,), jnp.float32)
    itab_vec = jnp.zeros((16,), jnp.int32)
    for l in range(_K):
        lsplat = jnp.full((16,), l, jnp.int32)
        row = plsc.load_gather(tab_v, [lsplat * _K + iota_c])
        m = jnp.max(jnp.where(valid, row, jnp.float32(-3e38)))
        e = jnp.where(valid, jnp.exp(row - m), jnp.float32(0.0))
        p = e / jnp.sum(e)
        pm = jnp.max(jnp.where(valid, p, jnp.float32(-1.0)))
        first = plsc.all_reduce_ffs((p == pm) & valid)
        onehot = jnp.where(iota == first, jnp.float32(1.0), jnp.float32(0.0))
        sel = (onehot - p) + p
        sdim_l = jnp.sum(sel * dims)
        fidx = lsplat * _K + iota
        plsc.store_scatter(ptab, [fidx], p, mask=valid)
        plsc.store_scatter(stab, [fidx], sel, mask=valid)
        sdim_vec = jnp.where(iota == l, sdim_l, sdim_vec)
        itab_vec = jnp.where(iota == l, first, itab_vec)
    dtab[pl.ds(0, 16)] = sdim_vec
    itab[pl.ds(0, 16)] = itab_vec

    def q_body(c, _):
        o = c * 16
        lvl = idx_v[pl.ds(o, 16)]
        dbuf[pl.ds(o, 16)] = plsc.load_gather(dtab, [lvl])
        iobuf[pl.ds(o, 16)] = plsc.load_gather(itab, [lvl])
        lvl6 = lvl * _K
        for k in range(_K):
            fidx = lvl6 + k
            ko = k * _QPW + o
            lbuf[pl.ds(ko, 16)] = plsc.load_gather(tab_v, [fidx])
            pbuf[pl.ds(ko, 16)] = plsc.load_gather(ptab, [fidx])
            sbuf[pl.ds(ko, 16)] = plsc.load_gather(stab, [fidx])
        return 0

    lax.fori_loop(0, _QPW // 16, q_body, 0)

    hs = []
    sems = [s0, s1, s2, s3]
    for k in range(_K):
        off = k * _B + base
        hs.append(pltpu.async_copy(
            sbuf.at[pl.ds(k * _QPW, _QPW)], sel_hbm.at[pl.ds(off, _QPW)],
            sems[k % 2]))
        hs.append(pltpu.async_copy(
            lbuf.at[pl.ds(k * _QPW, _QPW)], lg_hbm.at[pl.ds(off, _QPW)],
            sems[2 + k % 2]))
        hs.append(pltpu.async_copy(
            pbuf.at[pl.ds(k * _QPW, _QPW)], pr_hbm.at[pl.ds(off, _QPW)],
            sems[k % 2]))
    hs.append(pltpu.async_copy(dbuf, sdim_hbm.at[pl.ds(base, _QPW)], s2))
    hs.append(pltpu.async_copy(iobuf, io_hbm.at[pl.ds(base, _QPW)], s3))
    for h in hs:
        h.wait()


def _sc_call(bloom_levels, tab_flat):
    mesh = plsc.VectorSubcoreMesh(core_axis_name="c", subcore_axis_name="s")
    out_type = (
        jax.ShapeDtypeStruct((_B * _K,), jnp.float32),
        jax.ShapeDtypeStruct((_B,), jnp.float32),
        jax.ShapeDtypeStruct((_B * _K,), jnp.float32),
        jax.ShapeDtypeStruct((_B * _K,), jnp.float32),
        jax.ShapeDtypeStruct((_B,), jnp.int32),
    )
    scratch = [
        pltpu.VMEM((_QPW,), jnp.int32),
        pltpu.VMEM((128,), jnp.float32),
        pltpu.VMEM((128,), jnp.float32),
        pltpu.VMEM((128,), jnp.float32),
        pltpu.VMEM((128,), jnp.float32),
        pltpu.VMEM((128,), jnp.int32),
        pltpu.VMEM((_QPW * _K,), jnp.float32),
        pltpu.VMEM((_QPW,), jnp.float32),
        pltpu.VMEM((_QPW * _K,), jnp.float32),
        pltpu.VMEM((_QPW * _K,), jnp.float32),
        pltpu.VMEM((_QPW,), jnp.int32),
        pltpu.SemaphoreType.DMA,
        pltpu.SemaphoreType.DMA,
        pltpu.SemaphoreType.DMA,
        pltpu.SemaphoreType.DMA,
    ]
    return pl.kernel(
        _sc_body, out_type=out_type, mesh=mesh, scratch_types=scratch,
        compiler_params=pltpu.CompilerParams(needs_layout_passes=False),
    )(bloom_levels, tab_flat)


def _transpose_cols(src_ref, dst_ref, zeros2):
    for jj in range(_QPG // 128):
        chunk = src_ref[:, jj, :]
        c8 = jnp.concatenate([chunk, zeros2], axis=0)
        t = jnp.transpose(c8)
        dst_ref[pl.ds(jj * 128, 128), :] = t[:, :_K]


def _tc_body(tab_ref, lvl_ref, s3_ref, l3_ref, p3_ref,
             sel_ref, lg_ref, pr_ref, avg_ref, ent_ref, td_ref, cnt_ref):
    g = pl.program_id(0)
    zeros2 = jnp.zeros((8 - _K, 128), jnp.float32)
    _transpose_cols(s3_ref, sel_ref, zeros2)
    _transpose_cols(l3_ref, lg_ref, zeros2)
    _transpose_cols(p3_ref, pr_ref, zeros2)

    lvlb = lvl_ref[...]
    iota128 = lax.broadcasted_iota(jnp.int32, (1, 128), 1)
    c = jnp.zeros((1, 128), jnp.float32)
    for l in range(_K):
        cl = jnp.sum((lvlb == l).astype(jnp.float32))
        c = jnp.where(iota128 == l, cl, c)

    @pl.when(g == 0)
    def _init():
        cnt_ref[...] = c

    @pl.when(g > 0)
    def _acc():
        cnt_ref[...] = cnt_ref[...] + c

    @pl.when(g == _GRID - 1)
    def _finalize():
        lg = tab_ref[...]
        c6 = cnt_ref[...][:, :_K]
        m = jnp.max(lg, axis=1, keepdims=True)
        e = jnp.exp(lg - m)
        p = e / jnp.sum(e, axis=1, keepdims=True)
        iota2d = lax.broadcasted_iota(jnp.int32, (_K, _K), 1)
        dims2d = jnp.zeros((_K, _K), jnp.float32)
        for i, v in enumerate(_DIMS):
            dims2d = jnp.where(iota2d == i, jnp.float32(v), dims2d)
        td_ref[...] = jnp.sum(p * dims2d, axis=1, keepdims=True)
        ent_l = -jnp.sum(p * jnp.log(p + 1e-10), axis=1, keepdims=True)
        pm = jnp.max(p, axis=1, keepdims=True)
        first = jnp.min(jnp.where(p == pm, iota2d, _K), axis=1, keepdims=True)
        hard = (iota2d == first).astype(jnp.float32)
        sdim_l = jnp.sum(((hard - p) + p) * dims2d, axis=1, keepdims=True)
        inv_b = jnp.float32(1.0 / _B)
        avg_ref[...] = jnp.dot(c6, sdim_l,
                               preferred_element_type=jnp.float32) * inv_b
        ent_ref[...] = jnp.dot(c6, ent_l,
                               preferred_element_type=jnp.float32) * inv_b


def _tc_call(tab, lvl2d, sel3, lg3, pr3):
    blk3 = pl.BlockSpec((_K, _QPG // 128, 128), lambda g: (0, g, 0))
    blk_out = pl.BlockSpec((_QPG, _K), lambda g: (g, 0))
    return pl.pallas_call(
        _tc_body,
        grid=(_GRID,),
        in_specs=[
            pl.BlockSpec((_K, _K), lambda g: (0, 0)),
            pl.BlockSpec((_B // _GRID // 128, 128), lambda g: (g, 0)),
            blk3, blk3, blk3,
        ],
        out_specs=(
            blk_out, blk_out, blk_out,
            pl.BlockSpec((1, 1), lambda g: (0, 0)),
            pl.BlockSpec((1, 1), lambda g: (0, 0)),
            pl.BlockSpec((_K, 1), lambda g: (0, 0)),
        ),
        out_shape=(
            jax.ShapeDtypeStruct((_B, _K), jnp.float32),
            jax.ShapeDtypeStruct((_B, _K), jnp.float32),
            jax.ShapeDtypeStruct((_B, _K), jnp.float32),
            jax.ShapeDtypeStruct((1, 1), jnp.float32),
            jax.ShapeDtypeStruct((1, 1), jnp.float32),
            jax.ShapeDtypeStruct((_K, 1), jnp.float32),
        ),
        scratch_shapes=[pltpu.VMEM((1, 128), jnp.float32)],
    )(tab, lvl2d, sel3, lg3, pr3)


def kernel(bloom_levels, bloom_dim_logits):
    lvl = bloom_levels.astype(jnp.int32)
    tab = bloom_dim_logits.astype(jnp.float32)
    sel_c, sdim, lg_c, pr_c, idx = _sc_call(lvl, tab.reshape(_K * _K))
    selection, logits, probs, avg11, ent11, td61 = _tc_call(
        tab, lvl.reshape(128, 128),
        sel_c.reshape(_K, 128, 128),
        lg_c.reshape(_K, 128, 128),
        pr_c.reshape(_K, 128, 128),
    )
    avg_dim = avg11.reshape(())
    entropy = ent11.reshape(())
    table_dims = td61.reshape(_K)
    return (selection, sdim, avg_dim, entropy, table_dims, logits, probs, idx)


# speedup vs baseline: 5.1239x; 2.2975x over previous
import jax
import jax.numpy as jnp
from jax import lax
from jax.experimental import pallas as pl
from jax.experimental.pallas import tpu as pltpu
from jax.experimental.pallas import tpu_sc as plsc

_B = 16384
_K = 6
_NW = 32
_QPW = _B // _NW
_DIMS = (64.0, 128.0, 256.0, 384.0, 512.0, 768.0)


def _dims_vec(iota):
    d = jnp.zeros((16,), jnp.float32)
    for i, v in enumerate(_DIMS):
        d = jnp.where(iota == i, jnp.float32(v), d)
    return d


def _sc_body(lvl_hbm, tab_hbm, sel_hbm, sdim_hbm, lg_hbm, pr_hbm, io_hbm,
             idx_v, tab_v, ptab, stab, dtab, itab,
             sbuf, dbuf, lbuf, pbuf, iobuf, s0, s1, s2, s3):
    wid = lax.axis_index("s") * 2 + lax.axis_index("c")
    base = wid * _QPW

    h_idx = pltpu.async_copy(lvl_hbm.at[pl.ds(base, _QPW)], idx_v, s0)
    h_tab = pltpu.async_copy(tab_hbm, tab_v.at[pl.ds(0, _K * _K)], s1)
    h_idx.wait()
    h_tab.wait()

    iota = lax.iota(jnp.int32, 16)
    valid = iota < _K
    iota_c = jnp.where(valid, iota, _K - 1)
    dims = _dims_vec(iota)

    sdim_vec = jnp.zeros((16,), jnp.float32)
    itab_vec = jnp.zeros((16,), jnp.int32)
    for l in range(_K):
        lsplat = jnp.full((16,), l, jnp.int32)
        row = plsc.load_gather(tab_v, [lsplat * _K + iota_c])
        m = jnp.max(jnp.where(valid, row, jnp.float32(-3e38)))
        e = jnp.where(valid, jnp.exp(row - m), jnp.float32(0.0))
        p = e / jnp.sum(e)
        pm = jnp.max(jnp.where(valid, p, jnp.float32(-1.0)))
        first = plsc.all_reduce_ffs((p == pm) & valid)
        onehot = jnp.where(iota == first, jnp.float32(1.0), jnp.float32(0.0))
        sel = (onehot - p) + p
        sdim_l = jnp.sum(sel * dims)
        fidx = lsplat * _K + iota
        plsc.store_scatter(ptab, [fidx], p, mask=valid)
        plsc.store_scatter(stab, [fidx], sel, mask=valid)
        sdim_vec = jnp.where(iota == l, sdim_l, sdim_vec)
        itab_vec = jnp.where(iota == l, first, itab_vec)
    dtab[pl.ds(0, 16)] = sdim_vec
    itab[pl.ds(0, 16)] = itab_vec

    def q_body(c, _):
        o = c * 16
        lvl = idx_v[pl.ds(o, 16)]
        dbuf[pl.ds(o, 16)] = plsc.load_gather(dtab, [lvl])
        iobuf[pl.ds(o, 16)] = plsc.load_gather(itab, [lvl])
        lvl6 = lvl * _K
        for k in range(_K):
            fidx = lvl6 + k
            ko = k * _QPW + o
            lbuf[pl.ds(ko, 16)] = plsc.load_gather(tab_v, [fidx])
            pbuf[pl.ds(ko, 16)] = plsc.load_gather(ptab, [fidx])
            sbuf[pl.ds(ko, 16)] = plsc.load_gather(stab, [fidx])
        return 0

    lax.fori_loop(0, _QPW // 16, q_body, 0)

    hs = []
    sems = [s0, s1, s2, s3]
    for k in range(_K):
        hs.append(pltpu.async_copy(
            sbuf.at[pl.ds(k * _QPW, _QPW)],
            sel_hbm.at[k, pl.ds(base, _QPW)], sems[k % 2]))
        hs.append(pltpu.async_copy(
            lbuf.at[pl.ds(k * _QPW, _QPW)],
            lg_hbm.at[k, pl.ds(base, _QPW)], sems[2 + k % 2]))
        hs.append(pltpu.async_copy(
            pbuf.at[pl.ds(k * _QPW, _QPW)],
            pr_hbm.at[k, pl.ds(base, _QPW)], sems[k % 2]))
    hs.append(pltpu.async_copy(dbuf, sdim_hbm.at[pl.ds(base, _QPW)], s2))
    hs.append(pltpu.async_copy(iobuf, io_hbm.at[pl.ds(base, _QPW)], s3))
    for h in hs:
        h.wait()


def _sc_call(bloom_levels, tab_flat):
    mesh = plsc.VectorSubcoreMesh(core_axis_name="c", subcore_axis_name="s")
    out_type = (
        jax.ShapeDtypeStruct((_K, _B), jnp.float32),
        jax.ShapeDtypeStruct((_B,), jnp.float32),
        jax.ShapeDtypeStruct((_K, _B), jnp.float32),
        jax.ShapeDtypeStruct((_K, _B), jnp.float32),
        jax.ShapeDtypeStruct((_B,), jnp.int32),
    )
    scratch = [
        pltpu.VMEM((_QPW,), jnp.int32),
        pltpu.VMEM((128,), jnp.float32),
        pltpu.VMEM((128,), jnp.float32),
        pltpu.VMEM((128,), jnp.float32),
        pltpu.VMEM((128,), jnp.float32),
        pltpu.VMEM((128,), jnp.int32),
        pltpu.VMEM((_QPW * _K,), jnp.float32),
        pltpu.VMEM((_QPW,), jnp.float32),
        pltpu.VMEM((_QPW * _K,), jnp.float32),
        pltpu.VMEM((_QPW * _K,), jnp.float32),
        pltpu.VMEM((_QPW,), jnp.int32),
        pltpu.SemaphoreType.DMA,
        pltpu.SemaphoreType.DMA,
        pltpu.SemaphoreType.DMA,
        pltpu.SemaphoreType.DMA,
    ]
    return pl.kernel(
        _sc_body, out_type=out_type, mesh=mesh, scratch_types=scratch,
        compiler_params=pltpu.CompilerParams(needs_layout_passes=False),
    )(bloom_levels, tab_flat)


def _tc_body(tab_ref, lvl_ref, avg_ref, ent_ref, td_ref):
    lg = tab_ref[...]
    lvl2d = lvl_ref[...]
    m = jnp.max(lg, axis=1, keepdims=True)
    e = jnp.exp(lg - m)
    p = e / jnp.sum(e, axis=1, keepdims=True)
    iota2d = lax.broadcasted_iota(jnp.int32, (_K, _K), 1)
    dims2d = jnp.zeros((_K, _K), jnp.float32)
    for i, v in enumerate(_DIMS):
        dims2d = jnp.where(iota2d == i, jnp.float32(v), dims2d)
    td_ref[...] = jnp.sum(p * dims2d, axis=1, keepdims=True)
    ent_l = -jnp.sum(p * jnp.log(p + 1e-10), axis=1, keepdims=True)
    pm = jnp.max(p, axis=1, keepdims=True)
    first = jnp.min(jnp.where(p == pm, iota2d, _K), axis=1, keepdims=True)
    hard = (iota2d == first).astype(jnp.float32)
    sdim_l = jnp.sum(((hard - p) + p) * dims2d, axis=1, keepdims=True)
    iota16 = lax.broadcasted_iota(jnp.int32, (1, _K), 1)
    c6 = jnp.zeros((1, _K), jnp.float32)
    for l in range(_K):
        cl = jnp.sum((lvl2d == l).astype(jnp.float32))
        c6 = jnp.where(iota16 == l, cl, c6)
    inv_b = jnp.float32(1.0 / _B)
    avg_ref[...] = jnp.dot(c6, sdim_l, preferred_element_type=jnp.float32) * inv_b
    ent_ref[...] = jnp.dot(c6, ent_l, preferred_element_type=jnp.float32) * inv_b


def _tc_call(bloom_dim_logits, lvl2d):
    return pl.pallas_call(
        _tc_body,
        out_shape=(
            jax.ShapeDtypeStruct((1, 1), jnp.float32),
            jax.ShapeDtypeStruct((1, 1), jnp.float32),
            jax.ShapeDtypeStruct((_K, 1), jnp.float32),
        ),
    )(bloom_dim_logits, lvl2d)


def kernel(bloom_levels, bloom_dim_logits):
    lvl = bloom_levels.astype(jnp.int32)
    tab = bloom_dim_logits.astype(jnp.float32)
    sel_c, sdim, lg_c, pr_c, idx = _sc_call(lvl, tab.reshape(_K * _K))
    avg11, ent11, td61 = _tc_call(tab, lvl.reshape(128, 128))
    selection = jnp.transpose(sel_c)
    logits = jnp.transpose(lg_c)
    probs = jnp.transpose(pr_c)
    avg_dim = avg11.reshape(())
    entropy = ent11.reshape(())
    table_dims = td61.reshape(_K)
    return (selection, sdim, avg_dim, entropy, table_dims, logits, probs, idx)


# speedup vs baseline: 5.1743x; 1.0098x over previous
import jax
import jax.numpy as jnp
from jax import lax
from jax.experimental import pallas as pl
from jax.experimental.pallas import tpu as pltpu
from jax.experimental.pallas import tpu_sc as plsc

_B = 16384
_K = 6
_NW = 32
_QPW = _B // _NW
_DIMS = (64.0, 128.0, 256.0, 384.0, 512.0, 768.0)


def _dims_vec(iota):
    d = jnp.zeros((16,), jnp.float32)
    for i, v in enumerate(_DIMS):
        d = jnp.where(iota == i, jnp.float32(v), d)
    return d


def _sc_body(lvl_hbm, tab_hbm, sel_hbm, sdim_hbm, lg_hbm, pr_hbm, io_hbm,
             idx_v, tab_v, ptab, stab, dtab, itab,
             sbuf, dbuf, lbuf, pbuf, iobuf, s0, s1, s2, s3):
    wid = lax.axis_index("s") * 2 + lax.axis_index("c")
    base = wid * _QPW

    h_idx = pltpu.async_copy(lvl_hbm.at[pl.ds(base, _QPW)], idx_v, s0)
    h_tab = pltpu.async_copy(tab_hbm, tab_v.at[pl.ds(0, _K * _K)], s1)
    h_idx.wait()
    h_tab.wait()

    iota = lax.iota(jnp.int32, 16)
    valid = iota < _K
    iota_c = jnp.where(valid, iota, _K - 1)
    dims = _dims_vec(iota)

    sdim_vec = jnp.zeros((16,), jnp.float32)
    itab_vec = jnp.zeros((16,), jnp.int32)
    for l in range(_K):
        lsplat = jnp.full((16,), l, jnp.int32)
        row = plsc.load_gather(tab_v, [lsplat * _K + iota_c])
        m = jnp.max(jnp.where(valid, row, jnp.float32(-3e38)))
        e = jnp.where(valid, jnp.exp(row - m), jnp.float32(0.0))
        p = e / jnp.sum(e)
        pm = jnp.max(jnp.where(valid, p, jnp.float32(-1.0)))
        first = plsc.all_reduce_ffs((p == pm) & valid)
        onehot = jnp.where(iota == first, jnp.float32(1.0), jnp.float32(0.0))
        sel = (onehot - p) + p
        sdim_l = jnp.sum(sel * dims)
        fidx = lsplat * _K + iota
        plsc.store_scatter(ptab, [fidx], p, mask=valid)
        plsc.store_scatter(stab, [fidx], sel, mask=valid)
        sdim_vec = jnp.where(iota == l, sdim_l, sdim_vec)
        itab_vec = jnp.where(iota == l, first, itab_vec)
    dtab[pl.ds(0, 16)] = sdim_vec
    itab[pl.ds(0, 16)] = itab_vec

    def q_body(c, _):
        o = c * 16
        lvl = idx_v[pl.ds(o, 16)]
        dbuf[pl.ds(o, 16)] = plsc.load_gather(dtab, [lvl])
        iobuf[pl.ds(o, 16)] = plsc.load_gather(itab, [lvl])
        return 0

    lax.fori_loop(0, _QPW // 16, q_body, 0)
    hs = [pltpu.async_copy(dbuf, sdim_hbm.at[pl.ds(base, _QPW)], s3),
          pltpu.async_copy(iobuf, io_hbm.at[pl.ds(base, _QPW)], s3)]

    for k in range(_K):
        def k_body(c, _, k=k):
            o = c * 16
            fidx = idx_v[pl.ds(o, 16)] * _K + k
            ko = k * _QPW + o
            lbuf[pl.ds(ko, 16)] = plsc.load_gather(tab_v, [fidx])
            pbuf[pl.ds(ko, 16)] = plsc.load_gather(ptab, [fidx])
            sbuf[pl.ds(ko, 16)] = plsc.load_gather(stab, [fidx])
            return 0

        lax.fori_loop(0, _QPW // 16, k_body, 0)
        hs.append(pltpu.async_copy(
            sbuf.at[pl.ds(k * _QPW, _QPW)],
            sel_hbm.at[k, pl.ds(base, _QPW)], s0))
        hs.append(pltpu.async_copy(
            lbuf.at[pl.ds(k * _QPW, _QPW)],
            lg_hbm.at[k, pl.ds(base, _QPW)], s1))
        hs.append(pltpu.async_copy(
            pbuf.at[pl.ds(k * _QPW, _QPW)],
            pr_hbm.at[k, pl.ds(base, _QPW)], s2))
    for h in hs:
        h.wait()


def _sc_call(bloom_levels, tab_flat):
    mesh = plsc.VectorSubcoreMesh(core_axis_name="c", subcore_axis_name="s")
    out_type = (
        jax.ShapeDtypeStruct((_K, _B), jnp.float32),
        jax.ShapeDtypeStruct((_B,), jnp.float32),
        jax.ShapeDtypeStruct((_K, _B), jnp.float32),
        jax.ShapeDtypeStruct((_K, _B), jnp.float32),
        jax.ShapeDtypeStruct((_B,), jnp.int32),
    )
    scratch = [
        pltpu.VMEM((_QPW,), jnp.int32),
        pltpu.VMEM((128,), jnp.float32),
        pltpu.VMEM((128,), jnp.float32),
        pltpu.VMEM((128,), jnp.float32),
        pltpu.VMEM((128,), jnp.float32),
        pltpu.VMEM((128,), jnp.int32),
        pltpu.VMEM((_QPW * _K,), jnp.float32),
        pltpu.VMEM((_QPW,), jnp.float32),
        pltpu.VMEM((_QPW * _K,), jnp.float32),
        pltpu.VMEM((_QPW * _K,), jnp.float32),
        pltpu.VMEM((_QPW,), jnp.int32),
        pltpu.SemaphoreType.DMA,
        pltpu.SemaphoreType.DMA,
        pltpu.SemaphoreType.DMA,
        pltpu.SemaphoreType.DMA,
    ]
    return pl.kernel(
        _sc_body, out_type=out_type, mesh=mesh, scratch_types=scratch,
        compiler_params=pltpu.CompilerParams(needs_layout_passes=False),
    )(bloom_levels, tab_flat)


def _tc_body(tab_ref, lvl_ref, avg_ref, ent_ref, td_ref):
    lg = tab_ref[...]
    lvl1 = lvl_ref[...]
    m = jnp.max(lg, axis=1, keepdims=True)
    e = jnp.exp(lg - m)
    p = e / jnp.sum(e, axis=1, keepdims=True)
    iota2d = lax.broadcasted_iota(jnp.int32, (_K, _K), 1)
    dims2d = jnp.zeros((_K, _K), jnp.float32)
    for i, v in enumerate(_DIMS):
        dims2d = jnp.where(iota2d == i, jnp.float32(v), dims2d)
    td_ref[...] = jnp.sum(p * dims2d, axis=1, keepdims=True)
    ent_l = -jnp.sum(p * jnp.log(p + 1e-10), axis=1, keepdims=True)
    pm = jnp.max(p, axis=1, keepdims=True)
    first = jnp.min(jnp.where(p == pm, iota2d, _K), axis=1, keepdims=True)
    hard = (iota2d == first).astype(jnp.float32)
    sdim_l = jnp.sum(((hard - p) + p) * dims2d, axis=1, keepdims=True)
    iota16 = lax.broadcasted_iota(jnp.int32, (1, _K), 1)
    c6 = jnp.zeros((1, _K), jnp.float32)
    for l in range(_K):
        cl = jnp.sum((lvl1 == l).astype(jnp.float32))
        c6 = jnp.where(iota16 == l, cl, c6)
    inv_b = jnp.float32(1.0 / _B)
    avg_ref[...] = jnp.dot(c6, sdim_l, preferred_element_type=jnp.float32) * inv_b
    ent_ref[...] = jnp.dot(c6, ent_l, preferred_element_type=jnp.float32) * inv_b


def _tc_call(bloom_dim_logits, lvl1d):
    return pl.pallas_call(
        _tc_body,
        out_shape=(
            jax.ShapeDtypeStruct((1, 1), jnp.float32),
            jax.ShapeDtypeStruct((1, 1), jnp.float32),
            jax.ShapeDtypeStruct((_K, 1), jnp.float32),
        ),
    )(bloom_dim_logits, lvl1d)


def kernel(bloom_levels, bloom_dim_logits):
    lvl = bloom_levels.astype(jnp.int32)
    tab = bloom_dim_logits.astype(jnp.float32)
    sel_c, sdim, lg_c, pr_c, idx = _sc_call(lvl, tab.reshape(_K * _K))
    avg11, ent11, td61 = _tc_call(tab, lvl)
    selection = jnp.transpose(sel_c)
    logits = jnp.transpose(lg_c)
    probs = jnp.transpose(pr_c)
    avg_dim = avg11.reshape(())
    entropy = ent11.reshape(())
    table_dims = td61.reshape(_K)
    return (selection, sdim, avg_dim, entropy, table_dims, logits, probs, idx)
